# baseline (speedup 1.0000x reference)
import functools

import jax
import jax.numpy as jnp
from jax import lax
from jax.experimental import pallas as pl
from jax.experimental.pallas import tpu as pltpu
from jax.experimental.pallas import tpu_sc as plsc

NC = 2
NS = 16
LANES = 16
BBD = 128
BBM = 80


def _deg_body(nb, npad, per_tile, dst_hbm, ew_hbm, out_hbm, ewv, dstb, zbuf,
              acc, dsem, ssem):
    c = lax.axis_index("c")
    s = lax.axis_index("s")
    wid = c * NS + s
    gbase = wid * nb
    stripe = npad // NS
    soff = pl.multiple_of(s * stripe, 128)
    pltpu.sync_copy(ew_hbm.at[pl.ds(pl.multiple_of(wid * per_tile, 8),
                                    per_tile)], ewv)
    z = jnp.zeros((LANES,), jnp.float32)

    def zrow(r, _):
        zbuf[pl.ds(pl.multiple_of(r * LANES, LANES), LANES)] = z
        return 0

    lax.fori_loop(0, stripe // LANES, zrow, 0)
    pltpu.sync_copy(zbuf, acc.at[pl.ds(soff, stripe)])
    plsc.subcore_barrier()

    def fetch(b):
        sl = lax.rem(b, 4)
        pltpu.async_copy(dst_hbm.at[gbase + b], dstb.at[sl], dsem.at[sl])

    def ew_slice(b):
        return ewv.at[pl.ds(pl.multiple_of(b * BBD, 8), BBD)]

    def wait_scatter(b):
        pltpu.make_async_copy(ew_slice(b), acc.at[dstb.at[lax.rem(b, 4), 0]],
                              ssem.at[lax.rem(b, 2)]).wait()

    fetch(0)
    fetch(1)

    def batch(b, _):
        dsl = lax.rem(b, 4)
        pltpu.make_async_copy(dst_hbm.at[gbase + b], dstb.at[dsl],
                              dsem.at[dsl]).wait()

        @pl.when(b >= 2)
        def _():
            wait_scatter(b - 2)

        @pl.when(b + 2 < nb)
        def _():
            fetch(b + 2)

        pltpu.async_copy(ew_slice(b), acc.at[dstb.at[dsl, 0]],
                         ssem.at[lax.rem(b, 2)], add=True)
        return 0

    lax.fori_loop(0, nb, batch, 0)
    wait_scatter(nb - 2)
    wait_scatter(nb - 1)
    plsc.subcore_barrier()
    pltpu.sync_copy(acc.at[pl.ds(soff, stripe)], zbuf)
    pltpu.sync_copy(zbuf, out_hbm.at[c, pl.ds(soff, stripe)])


def _msg_body(nb, npad, d, hws_hbm, src_hbm, dst_hbm, ew_hbm,
              out_hbm, srcb, dstb, ewb, normv, rows3, acc, esem, gsem,
              ssem):
    c = lax.axis_index("c")
    s = lax.axis_index("s")
    wid = c * NS + s
    gbase = wid * nb
    stripe = npad // NS
    nchunk = stripe // BBM
    nvec = d // LANES
    z = jnp.zeros((LANES,), jnp.float32)

    def zrow(r, _):
        for j in range(nvec):
            rows3[0, r, pl.ds(j * LANES, LANES)] = z
        return 0

    lax.fori_loop(0, BBM, zrow, 0)
    for kk in range(nchunk):
        pltpu.sync_copy(
            rows3.at[0],
            acc.at[pl.ds(pl.multiple_of(s * stripe + kk * BBM, 8), BBM)])
    plsc.subcore_barrier()

    def fetch_edges(b):
        sl = lax.rem(b, 4)
        pltpu.async_copy(src_hbm.at[gbase + b], srcb.at[sl], esem.at[sl])
        pltpu.async_copy(dst_hbm.at[gbase + b], dstb.at[sl], esem.at[sl])
        pltpu.async_copy(ew_hbm.at[gbase + b], ewb.at[sl], esem.at[sl])

    def wait_edges(b):
        sl = lax.rem(b, 4)
        pltpu.make_async_copy(src_hbm.at[gbase + b], srcb.at[sl],
                              esem.at[sl]).wait()
        pltpu.make_async_copy(dst_hbm.at[gbase + b], dstb.at[sl],
                              esem.at[sl]).wait()
        pltpu.make_async_copy(ew_hbm.at[gbase + b], ewb.at[sl],
                              esem.at[sl]).wait()

    def issue_gather(b):
        pltpu.async_copy(hws_hbm.at[srcb.at[lax.rem(b, 4), 0]],
                         rows3.at[lax.rem(b, 3)], gsem.at[lax.rem(b, 3)])

    def wait_scatter(b):
        pltpu.make_async_copy(rows3.at[lax.rem(b, 3)],
                              acc.at[dstb.at[lax.rem(b, 4), 0]],
                              ssem.at[lax.rem(b, 3)]).wait()

    fetch_edges(0)
    fetch_edges(1)
    wait_edges(0)
    issue_gather(0)

    def batch(b, _):
        rslot = lax.rem(b, 3)
        eslot = lax.rem(b, 4)

        @pl.when(b + 1 < nb)
        def _():
            wait_edges(b + 1)

        @pl.when(b >= 2)
        def _():
            wait_scatter(b - 2)

        @pl.when(b + 1 < nb)
        def _():
            issue_gather(b + 1)

        pltpu.make_async_copy(hws_hbm.at[srcb.at[eslot, 0]], rows3.at[rslot],
                              gsem.at[rslot]).wait()
        for k in range(BBM // LANES):
            sl = pl.ds(k * LANES, LANES)
            normv[sl] = ewb[eslot, 0, sl]

        def scale(e4, _):
            for u in range(4):
                e = e4 * 4 + u
                sc = plsc.load_gather(normv, [jnp.broadcast_to(e, (LANES,))])
                for j2 in range(nvec):
                    sl2 = pl.ds(j2 * LANES, LANES)
                    rows3[rslot, e, sl2] = rows3[rslot, e, sl2] * sc
            return 0

        lax.fori_loop(0, BBM // 4, scale, 0)

        @pl.when(b + 2 < nb)
        def _():
            fetch_edges(b + 2)

        pltpu.async_copy(rows3.at[rslot], acc.at[dstb.at[eslot, 0]],
                         ssem.at[rslot], add=True)
        return 0

    lax.fori_loop(0, nb, batch, 0)
    wait_scatter(nb - 2)
    wait_scatter(nb - 1)
    plsc.subcore_barrier()
    for kk in range(nchunk):
        off = pl.multiple_of(s * stripe + kk * BBM, 8)
        pltpu.sync_copy(acc.at[pl.ds(off, BBM)], rows3.at[0])
        pltpu.sync_copy(rows3.at[0], out_hbm.at[c, pl.ds(off, BBM)])


def _dense_body(n, npad, d, h_ref, p_ref, w_ref, wih_ref, whh_ref, bih_ref,
                bhh_ref, pdegt_ref, hws_ref, dinv_ref, score_s, sel_s):
    f32 = jnp.float32
    p2 = p_ref[...]
    pn = jnp.sqrt(jnp.sum(p2 * p2, axis=1, keepdims=True))
    h = h_ref[...]
    sc0 = lax.dot_general(p2, h, (((1,), (1,)), ((), ())),
                          preferred_element_type=f32)
    score_s[...] = sc0 / pn
    iota = lax.broadcasted_iota(jnp.int32, (1, n), 1)

    def step(i, _):
        sv = score_s[...]
        m = jnp.max(sv, axis=1, keepdims=True)
        first = jnp.min(jnp.where(sv == m, iota, n), axis=1, keepdims=True)
        sel = iota == first
        t = jnp.tanh(m)
        sel_s[pl.ds(i, 1), :] = jnp.where(sel, t, 0.0).astype(f32)
        score_s[...] = jnp.where(sel, -jnp.inf, sv)
        return 0

    lax.fori_loop(0, d, step, 0)
    x = lax.dot_general(sel_s[...], h, (((1,), (0,)), ((), ())),
                        preferred_element_type=f32)
    gi = lax.dot_general(x, wih_ref[...], (((1,), (1,)), ((), ())),
                         preferred_element_type=f32) + bih_ref[...]
    gh = lax.dot_general(w_ref[...], whh_ref[...], (((1,), (1,)), ((), ())),
                         preferred_element_type=f32) + bhh_ref[...]
    i_r, i_z, i_n = gi[:, :d], gi[:, d:2 * d], gi[:, 2 * d:]
    h_r, h_z, h_n = gh[:, :d], gh[:, d:2 * d], gh[:, 2 * d:]
    r = jax.nn.sigmoid(i_r + h_r)
    zg = jax.nn.sigmoid(i_z + h_z)
    ng = jnp.tanh(i_n + r * h_n)
    wn = (1.0 - zg) * ng + zg * w_ref[...]
    hw = lax.dot_general(h, wn, (((1,), (0,)), ((), ())),
                         preferred_element_type=f32)
    pt = pdegt_ref[...]
    deg = 1.0 + pt[:, 0:1] + pt[:, 1:2]
    dinv = lax.rsqrt(deg)
    dinv_ref[...] = dinv
    hws_ref[0:n, :] = hw * dinv[0:n, :]
    hws_ref[n:npad, :] = jnp.zeros((npad - n, d), f32)


def _comb_body(parts_ref, hws_ref, dinv_ref, out_ref):
    p = parts_ref[...]
    out_ref[...] = (p[0] + p[1] + hws_ref[...]) * dinv_ref[...]


def kernel(H, edge_index, edge_weight, W, p, W_ih, W_hh, b_ih, b_hh):
    f32 = jnp.float32
    n, d = H.shape
    e = edge_weight.shape[0]
    nw = NC * NS
    nbd = (e + nw * BBD - 1) // (nw * BBD)
    per_tile_d = nbd * BBD
    e_pad = nw * per_tile_d
    nbm = e // (nw * BBM)
    npad = ((n + NS * BBM - 1) // (NS * BBM)) * (NS * BBM)

    dst_p = jnp.concatenate(
        [edge_index[1], jnp.zeros((e_pad - e,), jnp.int32)])
    ew_p = jnp.concatenate([edge_weight, jnp.zeros((e_pad - e,), f32)])
    dstd3 = dst_p.reshape(nw * nbd, 1, BBD)
    ew1 = ew_p

    mesh = plsc.VectorSubcoreMesh(core_axis_name="c", subcore_axis_name="s",
                                  num_cores=NC, num_subcores=NS)
    sc_params = pltpu.CompilerParams(needs_layout_passes=False)

    deg_call = pl.kernel(
        functools.partial(_deg_body, nbd, npad, per_tile_d),
        out_type=jax.ShapeDtypeStruct((NC, npad), f32),
        mesh=mesh,
        scratch_types=[
            pltpu.VMEM((per_tile_d,), f32),
            pltpu.VMEM((4, 1, BBD), jnp.int32),
            pltpu.VMEM((npad // NS,), f32),
            pltpu.VMEM_SHARED((npad,), f32),
            pltpu.SemaphoreType.DMA((4,)),
            pltpu.SemaphoreType.DMA((2,)),
        ],
        compiler_params=sc_params,
    )
    pdeg = deg_call(dstd3, ew1)
    pdegt = jnp.transpose(pdeg)

    dense_call = pl.pallas_call(
        functools.partial(_dense_body, n, npad, d),
        out_shape=[
            jax.ShapeDtypeStruct((npad, d), f32),
            jax.ShapeDtypeStruct((npad, 1), f32),
        ],
        scratch_shapes=[
            pltpu.VMEM((1, n), f32),
            pltpu.VMEM((d, n), f32),
        ],
    )
    hws, dinv = dense_call(H, p.reshape(1, d), W, W_ih, W_hh,
                           b_ih.reshape(1, 3 * d), b_hh.reshape(1, 3 * d),
                           pdegt)

    src3 = edge_index[0].reshape(nw * nbm, 1, BBM)
    dst3 = edge_index[1].reshape(nw * nbm, 1, BBM)
    ew3 = edge_weight.reshape(nw * nbm, 1, BBM)
    msg_call = pl.kernel(
        functools.partial(_msg_body, nbm, npad, d),
        out_type=jax.ShapeDtypeStruct((NC, npad, d), f32),
        mesh=mesh,
        scratch_types=[
            pltpu.VMEM((4, 1, BBM), jnp.int32),
            pltpu.VMEM((4, 1, BBM), jnp.int32),
            pltpu.VMEM((4, 1, BBM), f32),
            pltpu.VMEM((BBM,), f32),
            pltpu.VMEM((3, BBM, d), f32),
            pltpu.VMEM_SHARED((npad, d), f32),
            pltpu.SemaphoreType.DMA((4,)),
            pltpu.SemaphoreType.DMA((3,)),
            pltpu.SemaphoreType.DMA((3,)),
        ],
        compiler_params=sc_params,
    )
    parts = msg_call(hws, src3, dst3, ew3)

    rb = 400
    comb_call = pl.pallas_call(
        _comb_body,
        grid=(n // rb,),
        in_specs=[
            pl.BlockSpec((NC, rb, d), lambda i: (0, i, 0)),
            pl.BlockSpec((rb, d), lambda i: (i, 0)),
            pl.BlockSpec((rb, 1), lambda i: (i, 0)),
        ],
        out_specs=pl.BlockSpec((rb, d), lambda i: (i, 0)),
        out_shape=jax.ShapeDtypeStruct((n, d), f32),
    )
    return comb_call(parts, hws, dinv)


# speedup vs baseline: 1.0003x; 1.0003x over previous
import functools

import jax
import jax.numpy as jnp
from jax import lax
from jax.experimental import pallas as pl
from jax.experimental.pallas import tpu as pltpu
from jax.experimental.pallas import tpu_sc as plsc

NC = 2
NS = 16
LANES = 16
BBD = 128
BBM = 80


def _deg_body(nb, npad, per_tile, dst_hbm, ew_hbm, out_hbm, ewv, dstb, zbuf,
              acc, dsem, ssem):
    c = lax.axis_index("c")
    s = lax.axis_index("s")
    wid = c * NS + s
    gbase = wid * nb
    stripe = npad // NS
    soff = pl.multiple_of(s * stripe, 128)
    pltpu.sync_copy(ew_hbm.at[pl.ds(pl.multiple_of(wid * per_tile, 8),
                                    per_tile)], ewv)
    z = jnp.zeros((LANES,), jnp.float32)

    def zrow(r, _):
        zbuf[pl.ds(pl.multiple_of(r * LANES, LANES), LANES)] = z
        return 0

    lax.fori_loop(0, stripe // LANES, zrow, 0)
    pltpu.sync_copy(zbuf, acc.at[pl.ds(soff, stripe)])
    plsc.subcore_barrier()

    def fetch(b):
        sl = lax.rem(b, 4)
        pltpu.async_copy(dst_hbm.at[gbase + b], dstb.at[sl], dsem.at[sl])

    def ew_slice(b):
        return ewv.at[pl.ds(pl.multiple_of(b * BBD, 8), BBD)]

    def wait_scatter(b):
        pltpu.make_async_copy(ew_slice(b), acc.at[dstb.at[lax.rem(b, 4), 0]],
                              ssem.at[lax.rem(b, 2)]).wait()

    fetch(0)
    fetch(1)

    def batch(b, _):
        dsl = lax.rem(b, 4)
        pltpu.make_async_copy(dst_hbm.at[gbase + b], dstb.at[dsl],
                              dsem.at[dsl]).wait()

        @pl.when(b >= 2)
        def _():
            wait_scatter(b - 2)

        @pl.when(b + 2 < nb)
        def _():
            fetch(b + 2)

        pltpu.async_copy(ew_slice(b), acc.at[dstb.at[dsl, 0]],
                         ssem.at[lax.rem(b, 2)], add=True)
        return 0

    lax.fori_loop(0, nb, batch, 0)
    wait_scatter(nb - 2)
    wait_scatter(nb - 1)
    plsc.subcore_barrier()
    pltpu.sync_copy(acc.at[pl.ds(soff, stripe)], zbuf)
    pltpu.sync_copy(zbuf, out_hbm.at[c, pl.ds(soff, stripe)])


def _msg_body(nb, npad, d, hws_hbm, src_hbm, dst_hbm, ew_hbm,
              out_hbm, srcb, dstb, ewb, normv, rows3, acc, esem, gsem,
              ssem):
    c = lax.axis_index("c")
    s = lax.axis_index("s")
    wid = c * NS + s
    gbase = wid * nb
    stripe = npad // NS
    nchunk = stripe // BBM
    nvec = d // LANES
    z = jnp.zeros((LANES,), jnp.float32)

    def zrow(r, _):
        for j in range(nvec):
            rows3[0, r, pl.ds(j * LANES, LANES)] = z
        return 0

    lax.fori_loop(0, BBM, zrow, 0)
    for kk in range(nchunk):
        pltpu.sync_copy(
            rows3.at[0],
            acc.at[pl.ds(pl.multiple_of(s * stripe + kk * BBM, 8), BBM)])
    plsc.subcore_barrier()

    def fetch_edges(b):
        sl = lax.rem(b, 4)
        pltpu.async_copy(src_hbm.at[gbase + b], srcb.at[sl], esem.at[sl])
        pltpu.async_copy(dst_hbm.at[gbase + b], dstb.at[sl], esem.at[sl])
        pltpu.async_copy(ew_hbm.at[gbase + b], ewb.at[sl], esem.at[sl])

    def wait_edges(b):
        sl = lax.rem(b, 4)
        pltpu.make_async_copy(src_hbm.at[gbase + b], srcb.at[sl],
                              esem.at[sl]).wait()
        pltpu.make_async_copy(dst_hbm.at[gbase + b], dstb.at[sl],
                              esem.at[sl]).wait()
        pltpu.make_async_copy(ew_hbm.at[gbase + b], ewb.at[sl],
                              esem.at[sl]).wait()

    def issue_gather(b):
        pltpu.async_copy(hws_hbm.at[srcb.at[lax.rem(b, 4), 0]],
                         rows3.at[lax.rem(b, 3)], gsem.at[lax.rem(b, 3)])

    def wait_scatter(b):
        pltpu.make_async_copy(rows3.at[lax.rem(b, 3)],
                              acc.at[dstb.at[lax.rem(b, 4), 0]],
                              ssem.at[lax.rem(b, 3)]).wait()

    fetch_edges(0)
    fetch_edges(1)
    wait_edges(0)
    issue_gather(0)

    def batch(b, _):
        rslot = lax.rem(b, 3)
        eslot = lax.rem(b, 4)

        @pl.when(b + 1 < nb)
        def _():
            wait_edges(b + 1)
            issue_gather(b + 1)

        pltpu.make_async_copy(hws_hbm.at[srcb.at[eslot, 0]], rows3.at[rslot],
                              gsem.at[rslot]).wait()
        for k in range(BBM // LANES):
            sl = pl.ds(k * LANES, LANES)
            normv[sl] = ewb[eslot, 0, sl]

        def scale(e4, _):
            for u in range(4):
                e = e4 * 4 + u
                sc = plsc.load_gather(normv, [jnp.broadcast_to(e, (LANES,))])
                for j2 in range(nvec):
                    sl2 = pl.ds(j2 * LANES, LANES)
                    rows3[rslot, e, sl2] = rows3[rslot, e, sl2] * sc
            return 0

        lax.fori_loop(0, BBM // 4, scale, 0)

        @pl.when(b + 2 < nb)
        def _():
            fetch_edges(b + 2)

        @pl.when(b >= 1)
        def _():
            wait_scatter(b - 1)

        pltpu.async_copy(rows3.at[rslot], acc.at[dstb.at[eslot, 0]],
                         ssem.at[rslot], add=True)
        return 0

    lax.fori_loop(0, nb, batch, 0)
    wait_scatter(nb - 1)
    plsc.subcore_barrier()
    for kk in range(nchunk):
        off = pl.multiple_of(s * stripe + kk * BBM, 8)
        pltpu.sync_copy(acc.at[pl.ds(off, BBM)], rows3.at[0])
        pltpu.sync_copy(rows3.at[0], out_hbm.at[c, pl.ds(off, BBM)])


def _dense_body(n, npad, d, h_ref, p_ref, w_ref, wih_ref, whh_ref, bih_ref,
                bhh_ref, pdegt_ref, hws_ref, dinv_ref, score_s, sel_s):
    f32 = jnp.float32
    p2 = p_ref[...]
    pn = jnp.sqrt(jnp.sum(p2 * p2, axis=1, keepdims=True))
    h = h_ref[...]
    sc0 = lax.dot_general(p2, h, (((1,), (1,)), ((), ())),
                          preferred_element_type=f32)
    score_s[...] = sc0 / pn
    iota = lax.broadcasted_iota(jnp.int32, (1, n), 1)

    def step(i, _):
        sv = score_s[...]
        m = jnp.max(sv, axis=1, keepdims=True)
        first = jnp.min(jnp.where(sv == m, iota, n), axis=1, keepdims=True)
        sel = iota == first
        t = jnp.tanh(m)
        sel_s[pl.ds(i, 1), :] = jnp.where(sel, t, 0.0).astype(f32)
        score_s[...] = jnp.where(sel, -jnp.inf, sv)
        return 0

    lax.fori_loop(0, d, step, 0)
    x = lax.dot_general(sel_s[...], h, (((1,), (0,)), ((), ())),
                        preferred_element_type=f32)
    gi = lax.dot_general(x, wih_ref[...], (((1,), (1,)), ((), ())),
                         preferred_element_type=f32) + bih_ref[...]
    gh = lax.dot_general(w_ref[...], whh_ref[...], (((1,), (1,)), ((), ())),
                         preferred_element_type=f32) + bhh_ref[...]
    i_r, i_z, i_n = gi[:, :d], gi[:, d:2 * d], gi[:, 2 * d:]
    h_r, h_z, h_n = gh[:, :d], gh[:, d:2 * d], gh[:, 2 * d:]
    r = jax.nn.sigmoid(i_r + h_r)
    zg = jax.nn.sigmoid(i_z + h_z)
    ng = jnp.tanh(i_n + r * h_n)
    wn = (1.0 - zg) * ng + zg * w_ref[...]
    hw = lax.dot_general(h, wn, (((1,), (0,)), ((), ())),
                         preferred_element_type=f32)
    pt = pdegt_ref[...]
    deg = 1.0 + pt[:, 0:1] + pt[:, 1:2]
    dinv = lax.rsqrt(deg)
    dinv_ref[...] = dinv
    hws_ref[0:n, :] = hw * dinv[0:n, :]
    hws_ref[n:npad, :] = jnp.zeros((npad - n, d), f32)


def _comb_body(parts_ref, hws_ref, dinv_ref, out_ref):
    p = parts_ref[...]
    out_ref[...] = (p[0] + p[1] + hws_ref[...]) * dinv_ref[...]


def kernel(H, edge_index, edge_weight, W, p, W_ih, W_hh, b_ih, b_hh):
    f32 = jnp.float32
    n, d = H.shape
    e = edge_weight.shape[0]
    nw = NC * NS
    nbd = (e + nw * BBD - 1) // (nw * BBD)
    per_tile_d = nbd * BBD
    e_pad = nw * per_tile_d
    nbm = e // (nw * BBM)
    npad = ((n + NS * BBM - 1) // (NS * BBM)) * (NS * BBM)

    dst_p = jnp.concatenate(
        [edge_index[1], jnp.zeros((e_pad - e,), jnp.int32)])
    ew_p = jnp.concatenate([edge_weight, jnp.zeros((e_pad - e,), f32)])
    dstd3 = dst_p.reshape(nw * nbd, 1, BBD)
    ew1 = ew_p

    mesh = plsc.VectorSubcoreMesh(core_axis_name="c", subcore_axis_name="s",
                                  num_cores=NC, num_subcores=NS)
    sc_params = pltpu.CompilerParams(needs_layout_passes=False)

    deg_call = pl.kernel(
        functools.partial(_deg_body, nbd, npad, per_tile_d),
        out_type=jax.ShapeDtypeStruct((NC, npad), f32),
        mesh=mesh,
        scratch_types=[
            pltpu.VMEM((per_tile_d,), f32),
            pltpu.VMEM((4, 1, BBD), jnp.int32),
            pltpu.VMEM((npad // NS,), f32),
            pltpu.VMEM_SHARED((npad,), f32),
            pltpu.SemaphoreType.DMA((4,)),
            pltpu.SemaphoreType.DMA((2,)),
        ],
        compiler_params=sc_params,
    )
    pdeg = deg_call(dstd3, ew1)
    pdegt = jnp.transpose(pdeg)

    dense_call = pl.pallas_call(
        functools.partial(_dense_body, n, npad, d),
        out_shape=[
            jax.ShapeDtypeStruct((npad, d), f32),
            jax.ShapeDtypeStruct((npad, 1), f32),
        ],
        scratch_shapes=[
            pltpu.VMEM((1, n), f32),
            pltpu.VMEM((d, n), f32),
        ],
    )
    hws, dinv = dense_call(H, p.reshape(1, d), W, W_ih, W_hh,
                           b_ih.reshape(1, 3 * d), b_hh.reshape(1, 3 * d),
                           pdegt)

    src3 = edge_index[0].reshape(nw * nbm, 1, BBM)
    dst3 = edge_index[1].reshape(nw * nbm, 1, BBM)
    ew3 = edge_weight.reshape(nw * nbm, 1, BBM)
    msg_call = pl.kernel(
        functools.partial(_msg_body, nbm, npad, d),
        out_type=jax.ShapeDtypeStruct((NC, npad, d), f32),
        mesh=mesh,
        scratch_types=[
            pltpu.VMEM((4, 1, BBM), jnp.int32),
            pltpu.VMEM((4, 1, BBM), jnp.int32),
            pltpu.VMEM((4, 1, BBM), f32),
            pltpu.VMEM((BBM,), f32),
            pltpu.VMEM((3, BBM, d), f32),
            pltpu.VMEM_SHARED((npad, d), f32),
            pltpu.SemaphoreType.DMA((4,)),
            pltpu.SemaphoreType.DMA((3,)),
            pltpu.SemaphoreType.DMA((3,)),
        ],
        compiler_params=sc_params,
    )
    parts = msg_call(hws, src3, dst3, ew3)

    rb = 400
    comb_call = pl.pallas_call(
        _comb_body,
        grid=(n // rb,),
        in_specs=[
            pl.BlockSpec((NC, rb, d), lambda i: (0, i, 0)),
            pl.BlockSpec((rb, d), lambda i: (i, 0)),
            pl.BlockSpec((rb, 1), lambda i: (i, 0)),
        ],
        out_specs=pl.BlockSpec((rb, d), lambda i: (i, 0)),
        out_shape=jax.ShapeDtypeStruct((n, d), f32),
    )
    return comb_call(parts, hws, dinv)


# speedup vs baseline: 1.9123x; 1.9117x over previous
import functools

import jax
import jax.numpy as jnp
from jax import lax
from jax.experimental import pallas as pl
from jax.experimental.pallas import tpu as pltpu
from jax.experimental.pallas import tpu_sc as plsc

NC = 2
NS = 16
LANES = 16
BBD = 128
BBM = 80


def _deg_body(nb, npad, per_tile, dst_hbm, ew_hbm, out_hbm, ewv, dstb, zbuf,
              acc, dsem, ssem):
    c = lax.axis_index("c")
    s = lax.axis_index("s")
    wid = c * NS + s
    gbase = wid * nb
    stripe = npad // NS
    soff = pl.multiple_of(s * stripe, 128)
    pltpu.sync_copy(ew_hbm.at[pl.ds(pl.multiple_of(wid * per_tile, 8),
                                    per_tile)], ewv)
    z = jnp.zeros((LANES,), jnp.float32)

    def zrow(r, _):
        zbuf[pl.ds(pl.multiple_of(r * LANES, LANES), LANES)] = z
        return 0

    lax.fori_loop(0, stripe // LANES, zrow, 0)
    pltpu.sync_copy(zbuf, acc.at[pl.ds(soff, stripe)])
    plsc.subcore_barrier()

    def fetch(b):
        sl = lax.rem(b, 4)
        pltpu.async_copy(dst_hbm.at[gbase + b], dstb.at[sl], dsem.at[sl])

    def ew_slice(b):
        return ewv.at[pl.ds(pl.multiple_of(b * BBD, 8), BBD)]

    def wait_scatter(b):
        pltpu.make_async_copy(ew_slice(b), acc.at[dstb.at[lax.rem(b, 4), 0]],
                              ssem.at[lax.rem(b, 2)]).wait()

    fetch(0)
    fetch(1)

    def batch(b, _):
        dsl = lax.rem(b, 4)
        pltpu.make_async_copy(dst_hbm.at[gbase + b], dstb.at[dsl],
                              dsem.at[dsl]).wait()

        @pl.when(b >= 2)
        def _():
            wait_scatter(b - 2)

        @pl.when(b + 2 < nb)
        def _():
            fetch(b + 2)

        pltpu.async_copy(ew_slice(b), acc.at[dstb.at[dsl, 0]],
                         ssem.at[lax.rem(b, 2)], add=True)
        return 0

    lax.fori_loop(0, nb, batch, 0)
    wait_scatter(nb - 2)
    wait_scatter(nb - 1)
    plsc.subcore_barrier()
    pltpu.sync_copy(acc.at[pl.ds(soff, stripe)], zbuf)
    pltpu.sync_copy(zbuf, out_hbm.at[c, pl.ds(soff, stripe)])


def _msg_body(nb, npad, d, hws_hbm, src_hbm, dst_hbm, ew_hbm,
              out_hbm, srcb, dstb, ewb, normv, rows3, acc, esem, gsem,
              ssem):
    c = lax.axis_index("c")
    s = lax.axis_index("s")
    wid = c * NS + s
    gbase = wid * nb
    stripe = npad // NS
    nchunk = stripe // BBM
    nvec = d // LANES
    z = jnp.zeros((LANES,), jnp.float32)

    def zrow(r, _):
        for j in range(nvec):
            rows3[0, r, pl.ds(j * LANES, LANES)] = z
        return 0

    lax.fori_loop(0, BBM, zrow, 0)
    for kk in range(nchunk):
        pltpu.sync_copy(
            rows3.at[0],
            acc.at[pl.ds(pl.multiple_of(s * stripe + kk * BBM, 8), BBM)])
    plsc.subcore_barrier()

    def fetch_edges(b):
        sl = lax.rem(b, 3)
        pltpu.async_copy(src_hbm.at[gbase + b], srcb.at[sl], esem.at[sl])
        pltpu.async_copy(dst_hbm.at[gbase + b], dstb.at[sl], esem.at[sl])
        pltpu.async_copy(ew_hbm.at[gbase + b], ewb.at[sl], esem.at[sl])

    def wait_edges(b):
        sl = lax.rem(b, 3)
        pltpu.make_async_copy(src_hbm.at[gbase + b], srcb.at[sl],
                              esem.at[sl]).wait()
        pltpu.make_async_copy(dst_hbm.at[gbase + b], dstb.at[sl],
                              esem.at[sl]).wait()
        pltpu.make_async_copy(ew_hbm.at[gbase + b], ewb.at[sl],
                              esem.at[sl]).wait()

    def issue_gather(b, rslot):
        pltpu.async_copy(hws_hbm.at[srcb.at[lax.rem(b, 3), 0]],
                         rows3.at[rslot], gsem.at[rslot])

    fetch_edges(0)
    fetch_edges(1)
    wait_edges(0)
    issue_gather(0, 0)

    def batch(b, _):
        rslot = lax.rem(b, 2)
        other = 1 - rslot
        eslot = lax.rem(b, 3)

        @pl.when(b >= 1)
        def _():
            pltpu.make_async_copy(rows3.at[other],
                                  acc.at[dstb.at[lax.rem(b - 1, 3), 0]],
                                  ssem.at[other]).wait()

        @pl.when(b + 1 < nb)
        def _():
            wait_edges(b + 1)
            issue_gather(b + 1, other)

        pltpu.make_async_copy(hws_hbm.at[srcb.at[eslot, 0]], rows3.at[rslot],
                              gsem.at[rslot]).wait()
        for k in range(BBM // LANES):
            sl = pl.ds(k * LANES, LANES)
            normv[sl] = ewb[eslot, 0, sl]

        def scale(e4, _):
            for u in range(4):
                e = e4 * 4 + u
                sc = plsc.load_gather(normv, [jnp.broadcast_to(e, (LANES,))])
                for j2 in range(nvec):
                    sl2 = pl.ds(j2 * LANES, LANES)
                    rows3[rslot, e, sl2] = rows3[rslot, e, sl2] * sc
            return 0

        lax.fori_loop(0, BBM // 4, scale, 0)

        @pl.when(b + 2 < nb)
        def _():
            fetch_edges(b + 2)

        pltpu.async_copy(rows3.at[rslot], acc.at[dstb.at[eslot, 0]],
                         ssem.at[rslot], add=True)
        return 0

    lax.fori_loop(0, nb, batch, 0)
    lastslot = (nb - 1) % 2
    pltpu.make_async_copy(rows3.at[lastslot],
                          acc.at[dstb.at[(nb - 1) % 3, 0]],
                          ssem.at[lastslot]).wait()
    plsc.subcore_barrier()
    for kk in range(nchunk):
        off = pl.multiple_of(s * stripe + kk * BBM, 8)
        pltpu.sync_copy(acc.at[pl.ds(off, BBM)], rows3.at[0])
        pltpu.sync_copy(rows3.at[0], out_hbm.at[c, pl.ds(off, BBM)])


def _dense_body(n, npad, d, h_ref, p_ref, w_ref, wih_ref, whh_ref, bih_ref,
                bhh_ref, pdegt_ref, hws_ref, dinv_ref, score_s, sel_s):
    f32 = jnp.float32
    p2 = p_ref[...]
    pn = jnp.sqrt(jnp.sum(p2 * p2, axis=1, keepdims=True))
    h = h_ref[...]
    sc0 = lax.dot_general(p2, h, (((1,), (1,)), ((), ())),
                          preferred_element_type=f32)
    score_s[...] = sc0 / pn
    iota = lax.broadcasted_iota(jnp.int32, (1, n), 1)

    def step(i, _):
        sv = score_s[...]
        m = jnp.max(sv, axis=1, keepdims=True)
        first = jnp.min(jnp.where(sv == m, iota, n), axis=1, keepdims=True)
        sel = iota == first
        t = jnp.tanh(m)
        sel_s[pl.ds(i, 1), :] = jnp.where(sel, t, 0.0).astype(f32)
        score_s[...] = jnp.where(sel, -jnp.inf, sv)
        return 0

    lax.fori_loop(0, d, step, 0)
    x = lax.dot_general(sel_s[...], h, (((1,), (0,)), ((), ())),
                        preferred_element_type=f32)
    gi = lax.dot_general(x, wih_ref[...], (((1,), (1,)), ((), ())),
                         preferred_element_type=f32) + bih_ref[...]
    gh = lax.dot_general(w_ref[...], whh_ref[...], (((1,), (1,)), ((), ())),
                         preferred_element_type=f32) + bhh_ref[...]
    i_r, i_z, i_n = gi[:, :d], gi[:, d:2 * d], gi[:, 2 * d:]
    h_r, h_z, h_n = gh[:, :d], gh[:, d:2 * d], gh[:, 2 * d:]
    r = jax.nn.sigmoid(i_r + h_r)
    zg = jax.nn.sigmoid(i_z + h_z)
    ng = jnp.tanh(i_n + r * h_n)
    wn = (1.0 - zg) * ng + zg * w_ref[...]
    hw = lax.dot_general(h, wn, (((1,), (0,)), ((), ())),
                         preferred_element_type=f32)
    pt = pdegt_ref[...]
    deg = 1.0 + pt[:, 0:1] + pt[:, 1:2]
    dinv = lax.rsqrt(deg)
    dinv_ref[...] = dinv
    hws_ref[0:n, :] = hw * dinv[0:n, :]
    hws_ref[n:npad, :] = jnp.zeros((npad - n, d), f32)


def _comb_body(parts_ref, hws_ref, dinv_ref, out_ref):
    p = parts_ref[...]
    out_ref[...] = (p[0] + p[1] + hws_ref[...]) * dinv_ref[...]


def kernel(H, edge_index, edge_weight, W, p, W_ih, W_hh, b_ih, b_hh):
    f32 = jnp.float32
    n, d = H.shape
    e = edge_weight.shape[0]
    nw = NC * NS
    nbd = (e + nw * BBD - 1) // (nw * BBD)
    per_tile_d = nbd * BBD
    e_pad = nw * per_tile_d
    nbm = e // (nw * BBM)
    npad = ((n + NS * BBM - 1) // (NS * BBM)) * (NS * BBM)

    dst_p = jnp.concatenate(
        [edge_index[1], jnp.zeros((e_pad - e,), jnp.int32)])
    ew_p = jnp.concatenate([edge_weight, jnp.zeros((e_pad - e,), f32)])
    dstd3 = dst_p.reshape(nw * nbd, 1, BBD)
    ew1 = ew_p

    mesh = plsc.VectorSubcoreMesh(core_axis_name="c", subcore_axis_name="s",
                                  num_cores=NC, num_subcores=NS)
    sc_params = pltpu.CompilerParams(needs_layout_passes=False)

    deg_call = pl.kernel(
        functools.partial(_deg_body, nbd, npad, per_tile_d),
        out_type=jax.ShapeDtypeStruct((NC, npad), f32),
        mesh=mesh,
        scratch_types=[
            pltpu.VMEM((per_tile_d,), f32),
            pltpu.VMEM((4, 1, BBD), jnp.int32),
            pltpu.VMEM((npad // NS,), f32),
            pltpu.VMEM_SHARED((npad,), f32),
            pltpu.SemaphoreType.DMA((4,)),
            pltpu.SemaphoreType.DMA((2,)),
        ],
        compiler_params=sc_params,
    )
    pdeg = deg_call(dstd3, ew1)
    pdegt = jnp.transpose(pdeg)

    dense_call = pl.pallas_call(
        functools.partial(_dense_body, n, npad, d),
        out_shape=[
            jax.ShapeDtypeStruct((npad, d), f32),
            jax.ShapeDtypeStruct((npad, 1), f32),
        ],
        scratch_shapes=[
            pltpu.VMEM((1, n), f32),
            pltpu.VMEM((d, n), f32),
        ],
    )
    hws, dinv = dense_call(H, p.reshape(1, d), W, W_ih, W_hh,
                           b_ih.reshape(1, 3 * d), b_hh.reshape(1, 3 * d),
                           pdegt)

    src3 = edge_index[0].reshape(nw * nbm, 1, BBM)
    dst3 = edge_index[1].reshape(nw * nbm, 1, BBM)
    ew3 = edge_weight.reshape(nw * nbm, 1, BBM)
    msg_call = pl.kernel(
        functools.partial(_msg_body, nbm, npad, d),
        out_type=jax.ShapeDtypeStruct((NC, npad, d), f32),
        mesh=mesh,
        scratch_types=[
            pltpu.VMEM((3, 1, BBM), jnp.int32),
            pltpu.VMEM((3, 1, BBM), jnp.int32),
            pltpu.VMEM((3, 1, BBM), f32),
            pltpu.VMEM((BBM,), f32),
            pltpu.VMEM((2, BBM, d), f32),
            pltpu.VMEM_SHARED((npad, d), f32),
            pltpu.SemaphoreType.DMA((3,)),
            pltpu.SemaphoreType.DMA((2,)),
            pltpu.SemaphoreType.DMA((2,)),
        ],
        compiler_params=sc_params,
    )
    parts = msg_call(hws, src3, dst3, ew3)

    rb = 400
    comb_call = pl.pallas_call(
        _comb_body,
        grid=(n // rb,),
        in_specs=[
            pl.BlockSpec((NC, rb, d), lambda i: (0, i, 0)),
            pl.BlockSpec((rb, d), lambda i: (i, 0)),
            pl.BlockSpec((rb, 1), lambda i: (i, 0)),
        ],
        out_specs=pl.BlockSpec((rb, d), lambda i: (i, 0)),
        out_shape=jax.ShapeDtypeStruct((n, d), f32),
    )
    return comb_call(parts, hws, dinv)


# speedup vs baseline: 1.9226x; 1.0054x over previous
import functools

import jax
import jax.numpy as jnp
from jax import lax
from jax.experimental import pallas as pl
from jax.experimental.pallas import tpu as pltpu
from jax.experimental.pallas import tpu_sc as plsc

NC = 2
NS = 16
LANES = 16
BBD = 128
BBM = 80


def _deg_body(nb, npad, per_tile, dst_hbm, ew_hbm, out_hbm, ewv, dstb, zbuf,
              acc, dsem, ssem):
    c = lax.axis_index("c")
    s = lax.axis_index("s")
    wid = c * NS + s
    gbase = wid * nb
    stripe = npad // NS
    soff = pl.multiple_of(s * stripe, 128)
    pltpu.sync_copy(ew_hbm.at[pl.ds(pl.multiple_of(wid * per_tile, 8),
                                    per_tile)], ewv)
    z = jnp.zeros((LANES,), jnp.float32)

    def zrow(r, _):
        zbuf[pl.ds(pl.multiple_of(r * LANES, LANES), LANES)] = z
        return 0

    lax.fori_loop(0, stripe // LANES, zrow, 0)
    pltpu.sync_copy(zbuf, acc.at[pl.ds(soff, stripe)])
    plsc.subcore_barrier()

    def fetch(b):
        sl = lax.rem(b, 4)
        pltpu.async_copy(dst_hbm.at[gbase + b], dstb.at[sl], dsem.at[sl])

    def ew_slice(b):
        return ewv.at[pl.ds(pl.multiple_of(b * BBD, 8), BBD)]

    def wait_scatter(b):
        pltpu.make_async_copy(ew_slice(b), acc.at[dstb.at[lax.rem(b, 4), 0]],
                              ssem.at[lax.rem(b, 2)]).wait()

    fetch(0)
    fetch(1)

    def batch(b, _):
        dsl = lax.rem(b, 4)
        pltpu.make_async_copy(dst_hbm.at[gbase + b], dstb.at[dsl],
                              dsem.at[dsl]).wait()

        @pl.when(b >= 2)
        def _():
            wait_scatter(b - 2)

        @pl.when(b + 2 < nb)
        def _():
            fetch(b + 2)

        pltpu.async_copy(ew_slice(b), acc.at[dstb.at[dsl, 0]],
                         ssem.at[lax.rem(b, 2)], add=True)
        return 0

    lax.fori_loop(0, nb, batch, 0)
    wait_scatter(nb - 2)
    wait_scatter(nb - 1)
    plsc.subcore_barrier()
    pltpu.sync_copy(acc.at[pl.ds(soff, stripe)], zbuf)
    pltpu.sync_copy(zbuf, out_hbm.at[c, pl.ds(soff, stripe)])


def _msg_body(nb, npad, d, hws_hbm, src_hbm, dst_hbm, ew_hbm,
              out_hbm, srcb, dstb, ewb, rows3, acc, esem, gsem,
              ssem):
    c = lax.axis_index("c")
    s = lax.axis_index("s")
    wid = c * NS + s
    gbase = wid * nb
    stripe = npad // NS
    nchunk = stripe // BBM
    nvec = d // LANES
    z = jnp.zeros((LANES,), jnp.float32)

    def zrow(r, _):
        for j in range(nvec):
            rows3[0, r, pl.ds(j * LANES, LANES)] = z
        return 0

    lax.fori_loop(0, BBM, zrow, 0)
    for kk in range(nchunk):
        pltpu.sync_copy(
            rows3.at[0],
            acc.at[pl.ds(pl.multiple_of(s * stripe + kk * BBM, 8), BBM)])
    plsc.subcore_barrier()

    def fetch_edges(b):
        sl = lax.rem(b, 3)
        pltpu.async_copy(src_hbm.at[gbase + b], srcb.at[sl], esem.at[sl])
        pltpu.async_copy(dst_hbm.at[gbase + b], dstb.at[sl], esem.at[sl])
        pltpu.async_copy(ew_hbm.at[gbase + b], ewb.at[sl], esem.at[sl])

    def wait_edges(b):
        sl = lax.rem(b, 3)
        pltpu.make_async_copy(src_hbm.at[gbase + b], srcb.at[sl],
                              esem.at[sl]).wait()
        pltpu.make_async_copy(dst_hbm.at[gbase + b], dstb.at[sl],
                              esem.at[sl]).wait()
        pltpu.make_async_copy(ew_hbm.at[gbase + b], ewb.at[sl],
                              esem.at[sl]).wait()

    def issue_gather(b, rslot):
        pltpu.async_copy(hws_hbm.at[srcb.at[lax.rem(b, 3), 0]],
                         rows3.at[rslot], gsem.at[rslot])

    fetch_edges(0)
    fetch_edges(1)
    wait_edges(0)
    issue_gather(0, 0)

    def batch(b, _):
        rslot = lax.rem(b, 2)
        other = 1 - rslot
        eslot = lax.rem(b, 3)

        @pl.when(b >= 1)
        def _():
            pltpu.make_async_copy(rows3.at[other],
                                  acc.at[dstb.at[lax.rem(b - 1, 3), 0]],
                                  ssem.at[other]).wait()

        @pl.when(b + 1 < nb)
        def _():
            wait_edges(b + 1)
            issue_gather(b + 1, other)

        pltpu.make_async_copy(hws_hbm.at[srcb.at[eslot, 0]], rows3.at[rslot],
                              gsem.at[rslot]).wait()
        zi = jnp.zeros((LANES,), jnp.int32)
        esl16 = jnp.broadcast_to(eslot, (LANES,))

        def scale(e4, _):
            for u in range(4):
                e = e4 * 4 + u
                sc = plsc.load_gather(
                    ewb, [esl16, zi, jnp.broadcast_to(e, (LANES,))])
                for j2 in range(nvec):
                    sl2 = pl.ds(j2 * LANES, LANES)
                    rows3[rslot, e, sl2] = rows3[rslot, e, sl2] * sc
            return 0

        lax.fori_loop(0, BBM // 4, scale, 0)

        @pl.when(b + 2 < nb)
        def _():
            fetch_edges(b + 2)

        pltpu.async_copy(rows3.at[rslot], acc.at[dstb.at[eslot, 0]],
                         ssem.at[rslot], add=True)
        return 0

    lax.fori_loop(0, nb, batch, 0)
    lastslot = (nb - 1) % 2
    pltpu.make_async_copy(rows3.at[lastslot],
                          acc.at[dstb.at[(nb - 1) % 3, 0]],
                          ssem.at[lastslot]).wait()
    plsc.subcore_barrier()
    for kk in range(nchunk):
        off = pl.multiple_of(s * stripe + kk * BBM, 8)
        pltpu.sync_copy(acc.at[pl.ds(off, BBM)], rows3.at[0])
        pltpu.sync_copy(rows3.at[0], out_hbm.at[c, pl.ds(off, BBM)])


def _dense_body(n, npad, d, h_ref, p_ref, w_ref, wih_ref, whh_ref, bih_ref,
                bhh_ref, pdegt_ref, hws_ref, dinv_ref, score_s, sel_s):
    f32 = jnp.float32
    p2 = p_ref[...]
    pn = jnp.sqrt(jnp.sum(p2 * p2, axis=1, keepdims=True))
    h = h_ref[...]
    sc0 = lax.dot_general(p2, h, (((1,), (1,)), ((), ())),
                          preferred_element_type=f32)
    score_s[...] = sc0 / pn
    iota = lax.broadcasted_iota(jnp.int32, (1, n), 1)

    def step(i, _):
        sv = score_s[...]
        m = jnp.max(sv, axis=1, keepdims=True)
        first = jnp.min(jnp.where(sv == m, iota, n), axis=1, keepdims=True)
        sel = iota == first
        t = jnp.tanh(m)
        sel_s[pl.ds(i, 1), :] = jnp.where(sel, t, 0.0).astype(f32)
        score_s[...] = jnp.where(sel, -jnp.inf, sv)
        return 0

    lax.fori_loop(0, d, step, 0)
    x = lax.dot_general(sel_s[...], h, (((1,), (0,)), ((), ())),
                        preferred_element_type=f32)
    gi = lax.dot_general(x, wih_ref[...], (((1,), (1,)), ((), ())),
                         preferred_element_type=f32) + bih_ref[...]
    gh = lax.dot_general(w_ref[...], whh_ref[...], (((1,), (1,)), ((), ())),
                         preferred_element_type=f32) + bhh_ref[...]
    i_r, i_z, i_n = gi[:, :d], gi[:, d:2 * d], gi[:, 2 * d:]
    h_r, h_z, h_n = gh[:, :d], gh[:, d:2 * d], gh[:, 2 * d:]
    r = jax.nn.sigmoid(i_r + h_r)
    zg = jax.nn.sigmoid(i_z + h_z)
    ng = jnp.tanh(i_n + r * h_n)
    wn = (1.0 - zg) * ng + zg * w_ref[...]
    hw = lax.dot_general(h, wn, (((1,), (0,)), ((), ())),
                         preferred_element_type=f32)
    pt = pdegt_ref[...]
    deg = 1.0 + pt[:, 0:1] + pt[:, 1:2]
    dinv = lax.rsqrt(deg)
    dinv_ref[...] = dinv
    hws_ref[0:n, :] = hw * dinv[0:n, :]
    hws_ref[n:npad, :] = jnp.zeros((npad - n, d), f32)


def _comb_body(parts_ref, hws_ref, dinv_ref, out_ref):
    p = parts_ref[...]
    out_ref[...] = (p[0] + p[1] + hws_ref[...]) * dinv_ref[...]


def kernel(H, edge_index, edge_weight, W, p, W_ih, W_hh, b_ih, b_hh):
    f32 = jnp.float32
    n, d = H.shape
    e = edge_weight.shape[0]
    nw = NC * NS
    nbd = (e + nw * BBD - 1) // (nw * BBD)
    per_tile_d = nbd * BBD
    e_pad = nw * per_tile_d
    nbm = e // (nw * BBM)
    npad = ((n + NS * BBM - 1) // (NS * BBM)) * (NS * BBM)

    dst_p = jnp.concatenate(
        [edge_index[1], jnp.zeros((e_pad - e,), jnp.int32)])
    ew_p = jnp.concatenate([edge_weight, jnp.zeros((e_pad - e,), f32)])
    dstd3 = dst_p.reshape(nw * nbd, 1, BBD)
    ew1 = ew_p

    mesh = plsc.VectorSubcoreMesh(core_axis_name="c", subcore_axis_name="s",
                                  num_cores=NC, num_subcores=NS)
    sc_params = pltpu.CompilerParams(needs_layout_passes=False)

    deg_call = pl.kernel(
        functools.partial(_deg_body, nbd, npad, per_tile_d),
        out_type=jax.ShapeDtypeStruct((NC, npad), f32),
        mesh=mesh,
        scratch_types=[
            pltpu.VMEM((per_tile_d,), f32),
            pltpu.VMEM((4, 1, BBD), jnp.int32),
            pltpu.VMEM((npad // NS,), f32),
            pltpu.VMEM_SHARED((npad,), f32),
            pltpu.SemaphoreType.DMA((4,)),
            pltpu.SemaphoreType.DMA((2,)),
        ],
        compiler_params=sc_params,
    )
    pdeg = deg_call(dstd3, ew1)
    pdegt = jnp.transpose(pdeg)

    dense_call = pl.pallas_call(
        functools.partial(_dense_body, n, npad, d),
        out_shape=[
            jax.ShapeDtypeStruct((npad, d), f32),
            jax.ShapeDtypeStruct((npad, 1), f32),
        ],
        scratch_shapes=[
            pltpu.VMEM((1, n), f32),
            pltpu.VMEM((d, n), f32),
        ],
    )
    hws, dinv = dense_call(H, p.reshape(1, d), W, W_ih, W_hh,
                           b_ih.reshape(1, 3 * d), b_hh.reshape(1, 3 * d),
                           pdegt)

    src3 = edge_index[0].reshape(nw * nbm, 1, BBM)
    dst3 = edge_index[1].reshape(nw * nbm, 1, BBM)
    ew3 = edge_weight.reshape(nw * nbm, 1, BBM)
    msg_call = pl.kernel(
        functools.partial(_msg_body, nbm, npad, d),
        out_type=jax.ShapeDtypeStruct((NC, npad, d), f32),
        mesh=mesh,
        scratch_types=[
            pltpu.VMEM((3, 1, BBM), jnp.int32),
            pltpu.VMEM((3, 1, BBM), jnp.int32),
            pltpu.VMEM((3, 1, BBM), f32),
            pltpu.VMEM((2, BBM, d), f32),
            pltpu.VMEM_SHARED((npad, d), f32),
            pltpu.SemaphoreType.DMA((3,)),
            pltpu.SemaphoreType.DMA((2,)),
            pltpu.SemaphoreType.DMA((2,)),
        ],
        compiler_params=sc_params,
    )
    parts = msg_call(hws, src3, dst3, ew3)

    rb = 400
    comb_call = pl.pallas_call(
        _comb_body,
        grid=(n // rb,),
        in_specs=[
            pl.BlockSpec((NC, rb, d), lambda i: (0, i, 0)),
            pl.BlockSpec((rb, d), lambda i: (i, 0)),
            pl.BlockSpec((rb, 1), lambda i: (i, 0)),
        ],
        out_specs=pl.BlockSpec((rb, d), lambda i: (i, 0)),
        out_shape=jax.ShapeDtypeStruct((n, d), f32),
    )
    return comb_call(parts, hws, dinv)


# speedup vs baseline: 2.0062x; 1.0435x over previous
import functools

import jax
import jax.numpy as jnp
from jax import lax
from jax.experimental import pallas as pl
from jax.experimental.pallas import tpu as pltpu
from jax.experimental.pallas import tpu_sc as plsc

NC = 2
NS = 16
LANES = 16
BBD = 128
BBM = 80


def _deg_body(nb, npad, per_tile, dst_hbm, ew_hbm, out_hbm, ewv, dstb, zbuf,
              acc, dsem, ssem):
    c = lax.axis_index("c")
    s = lax.axis_index("s")
    wid = c * NS + s
    gbase = wid * nb
    stripe = npad // NS
    soff = pl.multiple_of(s * stripe, 128)
    pltpu.sync_copy(ew_hbm.at[pl.ds(pl.multiple_of(wid * per_tile, 8),
                                    per_tile)], ewv)
    z = jnp.zeros((LANES,), jnp.float32)

    def zrow(r, _):
        zbuf[pl.ds(pl.multiple_of(r * LANES, LANES), LANES)] = z
        return 0

    lax.fori_loop(0, stripe // LANES, zrow, 0)
    pltpu.sync_copy(zbuf, acc.at[pl.ds(soff, stripe)])
    plsc.subcore_barrier()

    def fetch(b):
        sl = lax.rem(b, 4)
        pltpu.async_copy(dst_hbm.at[gbase + b], dstb.at[sl], dsem.at[sl])

    def ew_slice(b):
        return ewv.at[pl.ds(pl.multiple_of(b * BBD, 8), BBD)]

    def wait_scatter(b):
        pltpu.make_async_copy(ew_slice(b), acc.at[dstb.at[lax.rem(b, 4), 0]],
                              ssem.at[lax.rem(b, 2)]).wait()

    fetch(0)
    fetch(1)

    def batch(b, _):
        dsl = lax.rem(b, 4)
        pltpu.make_async_copy(dst_hbm.at[gbase + b], dstb.at[dsl],
                              dsem.at[dsl]).wait()

        @pl.when(b >= 2)
        def _():
            wait_scatter(b - 2)

        @pl.when(b + 2 < nb)
        def _():
            fetch(b + 2)

        pltpu.async_copy(ew_slice(b), acc.at[dstb.at[dsl, 0]],
                         ssem.at[lax.rem(b, 2)], add=True)
        return 0

    lax.fori_loop(0, nb, batch, 0)
    wait_scatter(nb - 2)
    wait_scatter(nb - 1)
    plsc.subcore_barrier()
    pltpu.sync_copy(acc.at[pl.ds(soff, stripe)], zbuf)
    pltpu.sync_copy(zbuf, out_hbm.at[c, pl.ds(soff, stripe)])


def _msg_body(nb, npad, d, hws_hbm, src_hbm, dst_hbm, ew_hbm,
              out_hbm, srcb, dstb, ewb, rows3, acc, esem, gsem,
              ssem):
    c = lax.axis_index("c")
    s = lax.axis_index("s")
    wid = c * NS + s
    gbase = wid * nb
    stripe = npad // NS
    nchunk = stripe // BBM
    nvec = d // LANES
    z = jnp.zeros((LANES,), jnp.float32)

    def zrow(r, _):
        for j in range(nvec):
            rows3[0, r, pl.ds(j * LANES, LANES)] = z
        return 0

    lax.fori_loop(0, BBM, zrow, 0)
    for kk in range(nchunk):
        pltpu.sync_copy(
            rows3.at[0],
            acc.at[pl.ds(pl.multiple_of(s * stripe + kk * BBM, 8), BBM)])
    plsc.subcore_barrier()

    def fetch_edges(b):
        sl = lax.rem(b, 3)
        pltpu.async_copy(src_hbm.at[gbase + b], srcb.at[sl], esem.at[sl])
        pltpu.async_copy(dst_hbm.at[gbase + b], dstb.at[sl], esem.at[sl])
        pltpu.async_copy(ew_hbm.at[gbase + b], ewb.at[sl], esem.at[sl])

    def wait_edges(b):
        sl = lax.rem(b, 3)
        pltpu.make_async_copy(src_hbm.at[gbase + b], srcb.at[sl],
                              esem.at[sl]).wait()
        pltpu.make_async_copy(dst_hbm.at[gbase + b], dstb.at[sl],
                              esem.at[sl]).wait()
        pltpu.make_async_copy(ew_hbm.at[gbase + b], ewb.at[sl],
                              esem.at[sl]).wait()

    def issue_gather(b, rslot):
        pltpu.async_copy(hws_hbm.at[srcb.at[lax.rem(b, 3), 0]],
                         rows3.at[rslot], gsem.at[rslot])

    fetch_edges(0)
    fetch_edges(1)
    wait_edges(0)
    issue_gather(0, 0)

    def batch(b, _):
        rslot = lax.rem(b, 2)
        other = 1 - rslot
        eslot = lax.rem(b, 3)

        @pl.when(b >= 1)
        def _():
            pltpu.make_async_copy(rows3.at[other],
                                  acc.at[dstb.at[lax.rem(b - 1, 3), 0]],
                                  ssem.at[other]).wait()

        @pl.when(b + 1 < nb)
        def _():
            wait_edges(b + 1)
            issue_gather(b + 1, other)

        pltpu.make_async_copy(hws_hbm.at[srcb.at[eslot, 0]], rows3.at[rslot],
                              gsem.at[rslot]).wait()
        zi = jnp.zeros((LANES,), jnp.int32)
        esl16 = jnp.broadcast_to(eslot, (LANES,))

        def scale(e4, _):
            for u in range(4):
                e = e4 * 4 + u
                sc = plsc.load_gather(
                    ewb, [esl16, zi, jnp.broadcast_to(e, (LANES,))])
                for j2 in range(nvec):
                    sl2 = pl.ds(j2 * LANES, LANES)
                    rows3[rslot, e, sl2] = rows3[rslot, e, sl2] * sc
            return 0

        lax.fori_loop(0, BBM // 4, scale, 0)

        @pl.when(b + 2 < nb)
        def _():
            fetch_edges(b + 2)

        pltpu.async_copy(rows3.at[rslot], acc.at[dstb.at[eslot, 0]],
                         ssem.at[rslot], add=True)
        return 0

    lax.fori_loop(0, nb, batch, 0)
    lastslot = (nb - 1) % 2
    pltpu.make_async_copy(rows3.at[lastslot],
                          acc.at[dstb.at[(nb - 1) % 3, 0]],
                          ssem.at[lastslot]).wait()
    plsc.subcore_barrier()
    for kk in range(nchunk):
        off = pl.multiple_of(s * stripe + kk * BBM, 8)
        pltpu.sync_copy(acc.at[pl.ds(off, BBM)], rows3.at[0])
        pltpu.sync_copy(rows3.at[0], out_hbm.at[c, pl.ds(off, BBM)])


def _dense_body(n, npad, d, h_ref, p_ref, w_ref, wih_ref, whh_ref, bih_ref,
                bhh_ref, pdegt_ref, hws_ref, dinv_ref, score_s, sel_s):
    f32 = jnp.float32
    nl = npad // 8
    p2 = p_ref[...]
    pn = jnp.sqrt(jnp.sum(p2 * p2, axis=1, keepdims=True))
    h = h_ref[...]
    sc0 = lax.dot_general(p2, h, (((1,), (1,)), ((), ())),
                          preferred_element_type=f32)
    iota1 = lax.broadcasted_iota(jnp.int32, (1, npad), 1)
    sc1 = jnp.where(iota1 < n, sc0 / pn, -jnp.inf)
    for r in range(8):
        score_s[r:r + 1, :] = sc1[:, r * nl:(r + 1) * nl]
    iota2 = (lax.broadcasted_iota(jnp.int32, (8, nl), 0) * nl
             + lax.broadcasted_iota(jnp.int32, (8, nl), 1))

    def step(i, _):
        sv = score_s[...]
        m = jnp.max(jnp.max(sv, axis=1, keepdims=True), axis=0,
                    keepdims=True)
        first = jnp.min(jnp.min(jnp.where(sv == m, iota2, npad), axis=1,
                                keepdims=True), axis=0, keepdims=True)
        sel = iota2 == first
        t = jnp.tanh(m)
        v = jnp.where(sel, t, 0.0).astype(f32)
        for r in range(8):
            sel_s[pl.ds(i, 1), r * nl:(r + 1) * nl] = v[r:r + 1, :]
        score_s[...] = jnp.where(sel, -jnp.inf, sv)
        return 0

    lax.fori_loop(0, d, step, 0)
    x = lax.dot_general(sel_s[...], h, (((1,), (0,)), ((), ())),
                        preferred_element_type=f32)
    gi = lax.dot_general(x, wih_ref[...], (((1,), (1,)), ((), ())),
                         preferred_element_type=f32) + bih_ref[...]
    gh = lax.dot_general(w_ref[...], whh_ref[...], (((1,), (1,)), ((), ())),
                         preferred_element_type=f32) + bhh_ref[...]
    i_r, i_z, i_n = gi[:, :d], gi[:, d:2 * d], gi[:, 2 * d:]
    h_r, h_z, h_n = gh[:, :d], gh[:, d:2 * d], gh[:, 2 * d:]
    r = jax.nn.sigmoid(i_r + h_r)
    zg = jax.nn.sigmoid(i_z + h_z)
    ng = jnp.tanh(i_n + r * h_n)
    wn = (1.0 - zg) * ng + zg * w_ref[...]
    hw = lax.dot_general(h, wn, (((1,), (0,)), ((), ())),
                         preferred_element_type=f32)
    pt = pdegt_ref[...]
    deg = 1.0 + pt[:, 0:1] + pt[:, 1:2]
    dinv = lax.rsqrt(deg)
    dinv_ref[...] = dinv
    hws_ref[...] = hw * dinv


def _comb_body(parts_ref, hws_ref, dinv_ref, out_ref):
    p = parts_ref[...]
    out_ref[...] = (p[0] + p[1] + hws_ref[...]) * dinv_ref[...]


def kernel(H, edge_index, edge_weight, W, p, W_ih, W_hh, b_ih, b_hh):
    f32 = jnp.float32
    n, d = H.shape
    e = edge_weight.shape[0]
    nw = NC * NS
    nbd = (e + nw * BBD - 1) // (nw * BBD)
    per_tile_d = nbd * BBD
    e_pad = nw * per_tile_d
    nbm = e // (nw * BBM)
    npad = ((n + NS * BBM - 1) // (NS * BBM)) * (NS * BBM)

    dst_p = jnp.concatenate(
        [edge_index[1], jnp.zeros((e_pad - e,), jnp.int32)])
    ew_p = jnp.concatenate([edge_weight, jnp.zeros((e_pad - e,), f32)])
    dstd3 = dst_p.reshape(nw * nbd, 1, BBD)
    ew1 = ew_p

    mesh = plsc.VectorSubcoreMesh(core_axis_name="c", subcore_axis_name="s",
                                  num_cores=NC, num_subcores=NS)
    sc_params = pltpu.CompilerParams(needs_layout_passes=False)

    deg_call = pl.kernel(
        functools.partial(_deg_body, nbd, npad, per_tile_d),
        out_type=jax.ShapeDtypeStruct((NC, npad), f32),
        mesh=mesh,
        scratch_types=[
            pltpu.VMEM((per_tile_d,), f32),
            pltpu.VMEM((4, 1, BBD), jnp.int32),
            pltpu.VMEM((npad // NS,), f32),
            pltpu.VMEM_SHARED((npad,), f32),
            pltpu.SemaphoreType.DMA((4,)),
            pltpu.SemaphoreType.DMA((2,)),
        ],
        compiler_params=sc_params,
    )
    pdeg = deg_call(dstd3, ew1)
    pdegt = jnp.transpose(pdeg)

    dense_call = pl.pallas_call(
        functools.partial(_dense_body, n, npad, d),
        out_shape=[
            jax.ShapeDtypeStruct((npad, d), f32),
            jax.ShapeDtypeStruct((npad, 1), f32),
        ],
        scratch_shapes=[
            pltpu.VMEM((8, npad // 8), f32),
            pltpu.VMEM((d, npad), f32),
        ],
    )
    h_pad = jnp.pad(H, ((0, npad - n), (0, 0)))
    hws, dinv = dense_call(h_pad, p.reshape(1, d), W, W_ih, W_hh,
                           b_ih.reshape(1, 3 * d), b_hh.reshape(1, 3 * d),
                           pdegt)

    src3 = edge_index[0].reshape(nw * nbm, 1, BBM)
    dst3 = edge_index[1].reshape(nw * nbm, 1, BBM)
    ew3 = edge_weight.reshape(nw * nbm, 1, BBM)
    msg_call = pl.kernel(
        functools.partial(_msg_body, nbm, npad, d),
        out_type=jax.ShapeDtypeStruct((NC, npad, d), f32),
        mesh=mesh,
        scratch_types=[
            pltpu.VMEM((3, 1, BBM), jnp.int32),
            pltpu.VMEM((3, 1, BBM), jnp.int32),
            pltpu.VMEM((3, 1, BBM), f32),
            pltpu.VMEM((2, BBM, d), f32),
            pltpu.VMEM_SHARED((npad, d), f32),
            pltpu.SemaphoreType.DMA((3,)),
            pltpu.SemaphoreType.DMA((2,)),
            pltpu.SemaphoreType.DMA((2,)),
        ],
        compiler_params=sc_params,
    )
    parts = msg_call(hws, src3, dst3, ew3)

    rb = 400
    comb_call = pl.pallas_call(
        _comb_body,
        grid=(n // rb,),
        in_specs=[
            pl.BlockSpec((NC, rb, d), lambda i: (0, i, 0)),
            pl.BlockSpec((rb, d), lambda i: (i, 0)),
            pl.BlockSpec((rb, 1), lambda i: (i, 0)),
        ],
        out_specs=pl.BlockSpec((rb, d), lambda i: (i, 0)),
        out_shape=jax.ShapeDtypeStruct((n, d), f32),
    )
    return comb_call(parts, hws, dinv)


# speedup vs baseline: 2.4065x; 1.1995x over previous
import functools

import jax
import jax.numpy as jnp
from jax import lax
from jax.experimental import pallas as pl
from jax.experimental.pallas import tpu as pltpu
from jax.experimental.pallas import tpu_sc as plsc

NC = 2
NS = 16
LANES = 16
BBD = 128
BBM = 80


def _deg_body(nb, npad, per_tile, dst_hbm, ew_hbm, out_hbm, ewv, dstb, zbuf,
              acc, dsem, ssem):
    c = lax.axis_index("c")
    s = lax.axis_index("s")
    wid = c * NS + s
    gbase = wid * nb
    stripe = npad // NS
    soff = pl.multiple_of(s * stripe, 128)
    pltpu.sync_copy(ew_hbm.at[pl.ds(pl.multiple_of(wid * per_tile, 8),
                                    per_tile)], ewv)
    z = jnp.zeros((LANES,), jnp.float32)

    def zrow(r, _):
        zbuf[pl.ds(pl.multiple_of(r * LANES, LANES), LANES)] = z
        return 0

    lax.fori_loop(0, stripe // LANES, zrow, 0)
    pltpu.sync_copy(zbuf, acc.at[pl.ds(soff, stripe)])
    plsc.subcore_barrier()

    def fetch(b):
        sl = lax.rem(b, 4)
        pltpu.async_copy(dst_hbm.at[gbase + b], dstb.at[sl], dsem.at[sl])

    def ew_slice(b):
        return ewv.at[pl.ds(pl.multiple_of(b * BBD, 8), BBD)]

    def wait_scatter(b):
        pltpu.make_async_copy(ew_slice(b), acc.at[dstb.at[lax.rem(b, 4), 0]],
                              ssem.at[lax.rem(b, 2)]).wait()

    fetch(0)
    fetch(1)

    def batch(b, _):
        dsl = lax.rem(b, 4)
        pltpu.make_async_copy(dst_hbm.at[gbase + b], dstb.at[dsl],
                              dsem.at[dsl]).wait()

        @pl.when(b >= 2)
        def _():
            wait_scatter(b - 2)

        @pl.when(b + 2 < nb)
        def _():
            fetch(b + 2)

        pltpu.async_copy(ew_slice(b), acc.at[dstb.at[dsl, 0]],
                         ssem.at[lax.rem(b, 2)], add=True)
        return 0

    lax.fori_loop(0, nb, batch, 0)
    wait_scatter(nb - 2)
    wait_scatter(nb - 1)
    plsc.subcore_barrier()
    pltpu.sync_copy(acc.at[pl.ds(soff, stripe)], zbuf)
    pltpu.sync_copy(zbuf, out_hbm.at[c, pl.ds(soff, stripe)])


def _msg_body(nb, npad, d, hws_hbm, src_hbm, dst_hbm, ew_hbm,
              out_hbm, srcb, dstb, ewb, rows3, acc, esem, gsem,
              ssem):
    c = lax.axis_index("c")
    s = lax.axis_index("s")
    wid = c * NS + s
    gbase = wid * nb
    stripe = npad // NS
    nchunk = stripe // BBM
    nvec = d // LANES
    z = jnp.zeros((LANES,), jnp.float32)

    def zrow(r, _):
        for j in range(nvec):
            rows3[0, r, pl.ds(j * LANES, LANES)] = z
        return 0

    lax.fori_loop(0, BBM, zrow, 0)
    for kk in range(nchunk):
        pltpu.sync_copy(
            rows3.at[0],
            acc.at[pl.ds(pl.multiple_of(s * stripe + kk * BBM, 8), BBM)])
    plsc.subcore_barrier()

    def fetch_edges(b):
        sl = lax.rem(b, 3)
        pltpu.async_copy(src_hbm.at[gbase + b], srcb.at[sl], esem.at[sl])
        pltpu.async_copy(dst_hbm.at[gbase + b], dstb.at[sl], esem.at[sl])
        pltpu.async_copy(ew_hbm.at[gbase + b], ewb.at[sl], esem.at[sl])

    def wait_edges(b):
        sl = lax.rem(b, 3)
        pltpu.make_async_copy(src_hbm.at[gbase + b], srcb.at[sl],
                              esem.at[sl]).wait()
        pltpu.make_async_copy(dst_hbm.at[gbase + b], dstb.at[sl],
                              esem.at[sl]).wait()
        pltpu.make_async_copy(ew_hbm.at[gbase + b], ewb.at[sl],
                              esem.at[sl]).wait()

    def issue_gather(b, rslot):
        pltpu.async_copy(hws_hbm.at[srcb.at[lax.rem(b, 3), 0]],
                         rows3.at[rslot], gsem.at[rslot])

    fetch_edges(0)
    fetch_edges(1)
    wait_edges(0)
    issue_gather(0, 0)

    def batch(b, _):
        rslot = lax.rem(b, 2)
        other = 1 - rslot
        eslot = lax.rem(b, 3)

        @pl.when(b >= 1)
        def _():
            pltpu.make_async_copy(rows3.at[other],
                                  acc.at[dstb.at[lax.rem(b - 1, 3), 0]],
                                  ssem.at[other]).wait()

        @pl.when(b + 1 < nb)
        def _():
            wait_edges(b + 1)
            issue_gather(b + 1, other)

        pltpu.make_async_copy(hws_hbm.at[srcb.at[eslot, 0]], rows3.at[rslot],
                              gsem.at[rslot]).wait()
        zi = jnp.zeros((LANES,), jnp.int32)
        esl16 = jnp.broadcast_to(eslot, (LANES,))

        @functools.partial(plsc.parallel_loop, 0, BBM, unroll=4)
        def _(e):
            sc = plsc.load_gather(
                ewb, [esl16, zi, jnp.broadcast_to(e, (LANES,))])
            for j2 in range(nvec):
                sl2 = pl.ds(j2 * LANES, LANES)
                rows3[rslot, e, sl2] = rows3[rslot, e, sl2] * sc

        @pl.when(b + 2 < nb)
        def _():
            fetch_edges(b + 2)

        pltpu.async_copy(rows3.at[rslot], acc.at[dstb.at[eslot, 0]],
                         ssem.at[rslot], add=True)
        return 0

    lax.fori_loop(0, nb, batch, 0)
    lastslot = (nb - 1) % 2
    pltpu.make_async_copy(rows3.at[lastslot],
                          acc.at[dstb.at[(nb - 1) % 3, 0]],
                          ssem.at[lastslot]).wait()
    plsc.subcore_barrier()
    for kk in range(nchunk):
        off = pl.multiple_of(s * stripe + kk * BBM, 8)
        pltpu.sync_copy(acc.at[pl.ds(off, BBM)], rows3.at[0])
        pltpu.sync_copy(rows3.at[0], out_hbm.at[c, pl.ds(off, BBM)])


def _dense_body(n, npad, d, h_ref, p_ref, w_ref, wih_ref, whh_ref, bih_ref,
                bhh_ref, pdegt_ref, hws_ref, dinv_ref, score_s, sel_s):
    f32 = jnp.float32
    nl = npad // 8
    p2 = p_ref[...]
    pn = jnp.sqrt(jnp.sum(p2 * p2, axis=1, keepdims=True))
    h = h_ref[...]
    sc0 = lax.dot_general(p2, h, (((1,), (1,)), ((), ())),
                          preferred_element_type=f32)
    iota1 = lax.broadcasted_iota(jnp.int32, (1, npad), 1)
    sc1 = jnp.where(iota1 < n, sc0 / pn, -jnp.inf)
    for r in range(8):
        score_s[r:r + 1, :] = sc1[:, r * nl:(r + 1) * nl]
    iota2 = (lax.broadcasted_iota(jnp.int32, (8, nl), 0) * nl
             + lax.broadcasted_iota(jnp.int32, (8, nl), 1))

    def step(i, _):
        sv = score_s[...]
        m = jnp.max(jnp.max(sv, axis=1, keepdims=True), axis=0,
                    keepdims=True)
        first = jnp.min(jnp.min(jnp.where(sv == m, iota2, npad), axis=1,
                                keepdims=True), axis=0, keepdims=True)
        sel = iota2 == first
        t = jnp.tanh(m)
        v = jnp.where(sel, t, 0.0).astype(f32)
        for r in range(8):
            sel_s[pl.ds(i, 1), r * nl:(r + 1) * nl] = v[r:r + 1, :]
        score_s[...] = jnp.where(sel, -jnp.inf, sv)
        return 0

    lax.fori_loop(0, d, step, 0)
    x = lax.dot_general(sel_s[...], h, (((1,), (0,)), ((), ())),
                        preferred_element_type=f32)
    gi = lax.dot_general(x, wih_ref[...], (((1,), (1,)), ((), ())),
                         preferred_element_type=f32) + bih_ref[...]
    gh = lax.dot_general(w_ref[...], whh_ref[...], (((1,), (1,)), ((), ())),
                         preferred_element_type=f32) + bhh_ref[...]
    i_r, i_z, i_n = gi[:, :d], gi[:, d:2 * d], gi[:, 2 * d:]
    h_r, h_z, h_n = gh[:, :d], gh[:, d:2 * d], gh[:, 2 * d:]
    r = jax.nn.sigmoid(i_r + h_r)
    zg = jax.nn.sigmoid(i_z + h_z)
    ng = jnp.tanh(i_n + r * h_n)
    wn = (1.0 - zg) * ng + zg * w_ref[...]
    hw = lax.dot_general(h, wn, (((1,), (0,)), ((), ())),
                         preferred_element_type=f32)
    pt = pdegt_ref[...]
    deg = 1.0 + pt[:, 0:1] + pt[:, 1:2]
    dinv = lax.rsqrt(deg)
    dinv_ref[...] = dinv
    hws_ref[...] = hw * dinv


def _comb_body(parts_ref, hws_ref, dinv_ref, out_ref):
    p = parts_ref[...]
    out_ref[...] = (p[0] + p[1] + hws_ref[...]) * dinv_ref[...]


def kernel(H, edge_index, edge_weight, W, p, W_ih, W_hh, b_ih, b_hh):
    f32 = jnp.float32
    n, d = H.shape
    e = edge_weight.shape[0]
    nw = NC * NS
    nbd = (e + nw * BBD - 1) // (nw * BBD)
    per_tile_d = nbd * BBD
    e_pad = nw * per_tile_d
    nbm = e // (nw * BBM)
    npad = ((n + NS * BBM - 1) // (NS * BBM)) * (NS * BBM)

    dst_p = jnp.concatenate(
        [edge_index[1], jnp.zeros((e_pad - e,), jnp.int32)])
    ew_p = jnp.concatenate([edge_weight, jnp.zeros((e_pad - e,), f32)])
    dstd3 = dst_p.reshape(nw * nbd, 1, BBD)
    ew1 = ew_p

    mesh = plsc.VectorSubcoreMesh(core_axis_name="c", subcore_axis_name="s",
                                  num_cores=NC, num_subcores=NS)
    sc_params = pltpu.CompilerParams(needs_layout_passes=False)

    deg_call = pl.kernel(
        functools.partial(_deg_body, nbd, npad, per_tile_d),
        out_type=jax.ShapeDtypeStruct((NC, npad), f32),
        mesh=mesh,
        scratch_types=[
            pltpu.VMEM((per_tile_d,), f32),
            pltpu.VMEM((4, 1, BBD), jnp.int32),
            pltpu.VMEM((npad // NS,), f32),
            pltpu.VMEM_SHARED((npad,), f32),
            pltpu.SemaphoreType.DMA((4,)),
            pltpu.SemaphoreType.DMA((2,)),
        ],
        compiler_params=sc_params,
    )
    pdeg = deg_call(dstd3, ew1)
    pdegt = jnp.transpose(pdeg)

    dense_call = pl.pallas_call(
        functools.partial(_dense_body, n, npad, d),
        out_shape=[
            jax.ShapeDtypeStruct((npad, d), f32),
            jax.ShapeDtypeStruct((npad, 1), f32),
        ],
        scratch_shapes=[
            pltpu.VMEM((8, npad // 8), f32),
            pltpu.VMEM((d, npad), f32),
        ],
    )
    h_pad = jnp.pad(H, ((0, npad - n), (0, 0)))
    hws, dinv = dense_call(h_pad, p.reshape(1, d), W, W_ih, W_hh,
                           b_ih.reshape(1, 3 * d), b_hh.reshape(1, 3 * d),
                           pdegt)

    src3 = edge_index[0].reshape(nw * nbm, 1, BBM)
    dst3 = edge_index[1].reshape(nw * nbm, 1, BBM)
    ew3 = edge_weight.reshape(nw * nbm, 1, BBM)
    msg_call = pl.kernel(
        functools.partial(_msg_body, nbm, npad, d),
        out_type=jax.ShapeDtypeStruct((NC, npad, d), f32),
        mesh=mesh,
        scratch_types=[
            pltpu.VMEM((3, 1, BBM), jnp.int32),
            pltpu.VMEM((3, 1, BBM), jnp.int32),
            pltpu.VMEM((3, 1, BBM), f32),
            pltpu.VMEM((2, BBM, d), f32),
            pltpu.VMEM_SHARED((npad, d), f32),
            pltpu.SemaphoreType.DMA((3,)),
            pltpu.SemaphoreType.DMA((2,)),
            pltpu.SemaphoreType.DMA((2,)),
        ],
        compiler_params=sc_params,
    )
    parts = msg_call(hws, src3, dst3, ew3)

    rb = 400
    comb_call = pl.pallas_call(
        _comb_body,
        grid=(n // rb,),
        in_specs=[
            pl.BlockSpec((NC, rb, d), lambda i: (0, i, 0)),
            pl.BlockSpec((rb, d), lambda i: (i, 0)),
            pl.BlockSpec((rb, 1), lambda i: (i, 0)),
        ],
        out_specs=pl.BlockSpec((rb, d), lambda i: (i, 0)),
        out_shape=jax.ShapeDtypeStruct((n, d), f32),
    )
    return comb_call(parts, hws, dinv)
